# baseline (speedup 1.0000x reference)
import functools
import math

import jax
import jax.numpy as jnp
from jax.experimental import pallas as pl
from jax.experimental.pallas import tpu as pltpu

H = 12
D = 64
CR = 8
TOPK_RATIO = 0.05
QT = 256


def _qkv_kernel(x_ref, w_ref, b_ref, qnw_ref, knw_ref, q_ref, k_ref, v_ref):
    C = x_ref.shape[1]
    y = jnp.dot(x_ref[...], w_ref[...], preferred_element_type=jnp.float32)
    y = y + b_ref[...]
    yq = y[:, :C]
    yk = y[:, C:2 * C]

    def ln(t, w):
        mu = jnp.mean(t, axis=1, keepdims=True)
        var = jnp.mean((t - mu) ** 2, axis=1, keepdims=True)
        return (t - mu) * jax.lax.rsqrt(var + 1e-6) * w

    qnw = qnw_ref[...][:, :D]
    knw = knw_ref[...][:, :D]
    for h in range(H):
        q_ref[h, :, :] = ln(yq[:, h * D:(h + 1) * D], qnw)
        k_ref[h, :, :] = ln(yk[:, h * D:(h + 1) * D], knw)
        v_ref[h, :, :] = y[:, 2 * C + h * D:2 * C + (h + 1) * D]


def _softmax_rows(t):
    m = jnp.max(t, axis=1, keepdims=True)
    e = jnp.exp(t - m)
    return e / jnp.sum(e, axis=1, keepdims=True)


def _attn_kernel(q_ref, k_ref, v_ref, wp_ref, bp_ref, out_ref, *, n, k_sel):
    h = pl.program_id(1)
    nc = n // CR
    scale = D ** -0.5
    q = q_ref[0]
    k = k_ref[0]
    v = v_ref[0]

    kc = k[0:nc, :]
    for r in range(1, CR):
        kc = kc + k[r * nc:(r + 1) * nc, :]
    kc = kc * (1.0 / CR)

    dn = (((1,), (1,)), ((), ()))
    qs = q * scale
    router = jax.lax.dot_general(qs, kc, dn,
                                 preferred_element_type=jnp.float32)

    acc = router
    thresh = jnp.full((QT, 1), -jnp.inf, jnp.float32)
    need = jnp.full((QT, 1), float(k_sel), jnp.float32)
    for _ in range(k_sel):
        m = jnp.max(acc, axis=1, keepdims=True)
        eq = acc == m
        c = jnp.sum(jnp.where(eq, 1.0, 0.0), axis=1, keepdims=True)
        take = jnp.logical_and(need > 0.0, c >= need)
        thresh = jnp.where(take, m, thresh)
        need = need - c
        acc = jnp.where(eq, -jnp.inf, acc)

    bm = router >= thresh
    mask = jnp.concatenate([bm] * CR, axis=1)

    s = jax.lax.dot_general(qs, k, dn, preferred_element_type=jnp.float32)
    s = jnp.where(mask, s, -1e9)
    sm = jnp.max(s, axis=1, keepdims=True)
    p = jnp.exp(s - sm)
    o_sp = jnp.dot(p, v, preferred_element_type=jnp.float32)
    o_sp = o_sp / jnp.sum(p, axis=1, keepdims=True)

    phi_q = _softmax_rows(q)
    phi_k = _softmax_rows(k)
    wl = jax.lax.dot_general(phi_q, phi_k, dn,
                             preferred_element_type=jnp.float32)
    wl = jnp.where(mask, 0.0, wl)
    den = jnp.sum(wl, axis=1, keepdims=True) + 1e-6
    o_lin = jnp.dot(wl, v, preferred_element_type=jnp.float32) / den

    attn = o_sp + o_lin
    contrib = jnp.dot(attn, wp_ref[...], preferred_element_type=jnp.float32)

    @pl.when(h == 0)
    def _():
        out_ref[...] = bp_ref[...] + contrib

    @pl.when(h != 0)
    def _():
        out_ref[...] = out_ref[...] + contrib


def kernel(x, W_qkv, b_qkv, q_norm_w, k_norm_w, W_proj, b_proj):
    B, N, C = x.shape
    nt = N // QT
    x2 = x.reshape(N, C)

    q, k, v = pl.pallas_call(
        _qkv_kernel,
        grid=(nt,),
        in_specs=[
            pl.BlockSpec((QT, C), lambda i: (i, 0)),
            pl.BlockSpec((C, 3 * C), lambda i: (0, 0)),
            pl.BlockSpec((1, 3 * C), lambda i: (0, 0)),
            pl.BlockSpec((1, C), lambda i: (0, 0)),
            pl.BlockSpec((1, C), lambda i: (0, 0)),
        ],
        out_specs=[
            pl.BlockSpec((H, QT, D), lambda i: (0, i, 0)),
            pl.BlockSpec((H, QT, D), lambda i: (0, i, 0)),
            pl.BlockSpec((H, QT, D), lambda i: (0, i, 0)),
        ],
        out_shape=[jax.ShapeDtypeStruct((H, N, D), jnp.float32)] * 3,
    )(x2, W_qkv, b_qkv.reshape(1, 3 * C),
      jnp.tile(q_norm_w, H).reshape(1, C),
      jnp.tile(k_norm_w, H).reshape(1, C))

    nc = N // CR
    k_sel = max(1, int(math.ceil(TOPK_RATIO * nc)))
    kp = k.reshape(H, nc, CR, D).transpose(0, 2, 1, 3).reshape(H, N, D)
    vp = v.reshape(H, nc, CR, D).transpose(0, 2, 1, 3).reshape(H, N, D)

    out = pl.pallas_call(
        functools.partial(_attn_kernel, n=N, k_sel=k_sel),
        grid=(nt, H),
        in_specs=[
            pl.BlockSpec((1, QT, D), lambda i, h: (h, i, 0)),
            pl.BlockSpec((1, N, D), lambda i, h: (h, 0, 0)),
            pl.BlockSpec((1, N, D), lambda i, h: (h, 0, 0)),
            pl.BlockSpec((D, C), lambda i, h: (h, 0)),
            pl.BlockSpec((1, C), lambda i, h: (0, 0)),
        ],
        out_specs=pl.BlockSpec((QT, C), lambda i, h: (i, 0)),
        out_shape=jax.ShapeDtypeStruct((N, C), jnp.float32),
        compiler_params=pltpu.CompilerParams(
            dimension_semantics=("arbitrary", "arbitrary")),
    )(q, kp, vp, W_proj, b_proj.reshape(1, C))

    return out.reshape(B, N, C)


# speedup vs baseline: 1.0793x; 1.0793x over previous
import functools
import math

import jax
import jax.numpy as jnp
from jax.experimental import pallas as pl
from jax.experimental.pallas import tpu as pltpu

H = 12
D = 64
CR = 8
TOPK_RATIO = 0.05
QT = 256


def _softmax_rows(t):
    m = jnp.max(t, axis=1, keepdims=True)
    e = jnp.exp(t - m)
    return e / jnp.sum(e, axis=1, keepdims=True)


def _qkv_kernel(x_ref, w_ref, b_ref, qnw_ref, knw_ref,
                q_ref, k_ref, v_ref, pk_ref):
    C = x_ref.shape[1]
    y = jnp.dot(x_ref[...], w_ref[...], preferred_element_type=jnp.float32)
    y = y + b_ref[...]
    yq = y[:, :C]
    yk = y[:, C:2 * C]

    def ln(t, w):
        mu = jnp.mean(t, axis=1, keepdims=True)
        var = jnp.mean((t - mu) ** 2, axis=1, keepdims=True)
        return (t - mu) * jax.lax.rsqrt(var + 1e-6) * w

    qnw = qnw_ref[...]
    knw = knw_ref[...]
    for h in range(H):
        kh = ln(yk[:, h * D:(h + 1) * D], knw)
        q_ref[h, :, :] = ln(yq[:, h * D:(h + 1) * D], qnw)
        k_ref[h, :, :] = kh
        v_ref[h, :, :] = y[:, 2 * C + h * D:2 * C + (h + 1) * D]
        pk_ref[h, :, :] = _softmax_rows(kh)


def _attn_kernel(q_ref, k_ref, v_ref, pk_ref, wp_ref, bp_ref, out_ref,
                 acc_ref, *, n, k_sel):
    h = pl.program_id(1)
    nc = n // CR
    scale = D ** -0.5
    q = q_ref[0]
    k = k_ref[0]
    v = v_ref[0]
    phi_k = pk_ref[0]

    kc = k[0:nc, :]
    for r in range(1, CR):
        kc = kc + k[r * nc:(r + 1) * nc, :]
    kc = kc * (1.0 / CR)

    dn = (((1,), (1,)), ((), ()))
    qs = q * scale
    router = jax.lax.dot_general(qs, kc, dn,
                                 preferred_element_type=jnp.float32)

    vals = router
    thresh = jnp.full((QT, 1), jnp.inf, jnp.float32)
    need = jnp.full((QT, 1), float(k_sel), jnp.float32)
    for _ in range(k_sel):
        m = jnp.max(vals, axis=1, keepdims=True)
        eq = vals == m
        c = jnp.sum(jnp.where(eq, 1.0, 0.0), axis=1, keepdims=True)
        thresh = jnp.minimum(thresh, jnp.where(need > 0.0, m, jnp.inf))
        need = need - c
        vals = jnp.where(eq, -jnp.inf, vals)

    bm = router >= thresh
    mask = jnp.concatenate([bm] * CR, axis=1)

    s = jax.lax.dot_general(qs, k, dn, preferred_element_type=jnp.float32)
    s = jnp.where(mask, s, -1e9)
    sm = jnp.max(s, axis=1, keepdims=True)
    p = jnp.exp(s - sm)
    o_sp = jnp.dot(p, v, preferred_element_type=jnp.float32)
    o_sp = o_sp / jnp.sum(p, axis=1, keepdims=True)

    phi_q = _softmax_rows(q)
    wl = jax.lax.dot_general(phi_q, phi_k, dn,
                             preferred_element_type=jnp.float32)
    wl = jnp.where(mask, 0.0, wl)
    den = jnp.sum(wl, axis=1, keepdims=True) + 1e-6
    o_lin = jnp.dot(wl, v, preferred_element_type=jnp.float32) / den

    acc_ref[h] = o_sp + o_lin

    @pl.when(h == H - 1)
    def _():
        res = bp_ref[...]
        for hh in range(H):
            res = res + jnp.dot(acc_ref[hh], wp_ref[hh],
                                preferred_element_type=jnp.float32)
        out_ref[...] = res


def kernel(x, W_qkv, b_qkv, q_norm_w, k_norm_w, W_proj, b_proj):
    B, N, C = x.shape
    nt = N // QT
    x2 = x.reshape(N, C)

    q, k, v, pk = pl.pallas_call(
        _qkv_kernel,
        grid=(nt,),
        in_specs=[
            pl.BlockSpec((QT, C), lambda i: (i, 0)),
            pl.BlockSpec((C, 3 * C), lambda i: (0, 0)),
            pl.BlockSpec((1, 3 * C), lambda i: (0, 0)),
            pl.BlockSpec((1, D), lambda i: (0, 0)),
            pl.BlockSpec((1, D), lambda i: (0, 0)),
        ],
        out_specs=[
            pl.BlockSpec((H, QT, D), lambda i: (0, i, 0)),
            pl.BlockSpec((H, QT, D), lambda i: (0, i, 0)),
            pl.BlockSpec((H, QT, D), lambda i: (0, i, 0)),
            pl.BlockSpec((H, QT, D), lambda i: (0, i, 0)),
        ],
        out_shape=[jax.ShapeDtypeStruct((H, N, D), jnp.float32)] * 4,
    )(x2, W_qkv, b_qkv.reshape(1, 3 * C), q_norm_w.reshape(1, D),
      k_norm_w.reshape(1, D))

    nc = N // CR
    k_sel = max(1, int(math.ceil(TOPK_RATIO * nc)))
    kp = k.reshape(H, nc, CR, D).transpose(0, 2, 1, 3).reshape(H, N, D)
    vp = v.reshape(H, nc, CR, D).transpose(0, 2, 1, 3).reshape(H, N, D)
    pkp = pk.reshape(H, nc, CR, D).transpose(0, 2, 1, 3).reshape(H, N, D)

    out = pl.pallas_call(
        functools.partial(_attn_kernel, n=N, k_sel=k_sel),
        grid=(nt, H),
        in_specs=[
            pl.BlockSpec((1, QT, D), lambda i, h: (h, i, 0)),
            pl.BlockSpec((1, N, D), lambda i, h: (h, 0, 0)),
            pl.BlockSpec((1, N, D), lambda i, h: (h, 0, 0)),
            pl.BlockSpec((1, N, D), lambda i, h: (h, 0, 0)),
            pl.BlockSpec((H, D, C), lambda i, h: (0, 0, 0)),
            pl.BlockSpec((1, C), lambda i, h: (0, 0)),
        ],
        out_specs=pl.BlockSpec((QT, C), lambda i, h: (i, 0)),
        out_shape=jax.ShapeDtypeStruct((N, C), jnp.float32),
        scratch_shapes=[pltpu.VMEM((H, QT, D), jnp.float32)],
        compiler_params=pltpu.CompilerParams(
            dimension_semantics=("arbitrary", "arbitrary")),
    )(q, kp, vp, pkp, W_proj.reshape(H, D, C), b_proj.reshape(1, C))

    return out.reshape(B, N, C)


# speedup vs baseline: 1.2016x; 1.1133x over previous
import functools
import math

import jax
import jax.numpy as jnp
from jax.experimental import pallas as pl
from jax.experimental.pallas import tpu as pltpu

H = 12
D = 64
CR = 8
TOPK_RATIO = 0.05
QT = 512


def _softmax_rows(t):
    m = jnp.max(t, axis=1, keepdims=True)
    e = jnp.exp(t - m)
    return e / jnp.sum(e, axis=1, keepdims=True)


def _qkv_kernel(x_ref, w_ref, b_ref, qnw_ref, knw_ref,
                q_ref, k_ref, v_ref, pk_ref):
    C = x_ref.shape[1]
    y = jnp.dot(x_ref[...], w_ref[...], preferred_element_type=jnp.float32)
    y = y + b_ref[...]
    yq = y[:, :C]
    yk = y[:, C:2 * C]

    def ln(t, w):
        mu = jnp.mean(t, axis=1, keepdims=True)
        var = jnp.mean((t - mu) ** 2, axis=1, keepdims=True)
        return (t - mu) * jax.lax.rsqrt(var + 1e-6) * w

    qnw = qnw_ref[...]
    knw = knw_ref[...]
    for h in range(H):
        kh = ln(yk[:, h * D:(h + 1) * D], knw)
        q_ref[h, :, :] = ln(yq[:, h * D:(h + 1) * D], qnw)
        k_ref[h, :, :] = kh
        v_ref[h, :, :] = y[:, 2 * C + h * D:2 * C + (h + 1) * D]
        pk_ref[h, :, :] = _softmax_rows(kh)


def _attn_kernel(q_ref, k_ref, v_ref, pk_ref, wp_ref, bp_ref, out_ref,
                 acc_ref, *, n, k_sel):
    h = pl.program_id(1)
    nc = n // CR
    scale = D ** -0.5
    q = q_ref[0]
    k = k_ref[0]
    v = v_ref[0]
    phi_k = pk_ref[0]

    kc = k[0:nc, :]
    for r in range(1, CR):
        kc = kc + k[r * nc:(r + 1) * nc, :]
    kc = kc * (1.0 / CR)

    dn = (((1,), (1,)), ((), ()))
    qs = q * scale
    router = jax.lax.dot_general(qs, kc, dn,
                                 preferred_element_type=jnp.float32)

    vals = router
    thresh = jnp.full((QT, 1), jnp.inf, jnp.float32)
    need = jnp.full((QT, 1), float(k_sel), jnp.float32)
    for _ in range(k_sel):
        m = jnp.max(vals, axis=1, keepdims=True)
        eq = vals == m
        c = jnp.sum(jnp.where(eq, 1.0, 0.0), axis=1, keepdims=True)
        thresh = jnp.minimum(thresh, jnp.where(need > 0.0, m, jnp.inf))
        need = need - c
        vals = jnp.where(eq, -jnp.inf, vals)

    bm = router >= thresh
    mask = jnp.concatenate([bm] * CR, axis=1)

    s = jax.lax.dot_general(qs, k, dn, preferred_element_type=jnp.float32)
    s = jnp.where(mask, s, -1e9)
    sm = jnp.max(s, axis=1, keepdims=True)
    p = jnp.exp(s - sm)
    o_sp = jnp.dot(p, v, preferred_element_type=jnp.float32)
    o_sp = o_sp / jnp.sum(p, axis=1, keepdims=True)

    phi_q = _softmax_rows(q)
    wl = jax.lax.dot_general(phi_q, phi_k, dn,
                             preferred_element_type=jnp.float32)
    wl = jnp.where(mask, 0.0, wl)
    den = jnp.sum(wl, axis=1, keepdims=True) + 1e-6
    o_lin = jnp.dot(wl, v, preferred_element_type=jnp.float32) / den

    acc_ref[h] = o_sp + o_lin

    @pl.when(h == H - 1)
    def _():
        res = bp_ref[...]
        for hh in range(H):
            res = res + jnp.dot(acc_ref[hh], wp_ref[hh],
                                preferred_element_type=jnp.float32)
        out_ref[...] = res


def kernel(x, W_qkv, b_qkv, q_norm_w, k_norm_w, W_proj, b_proj):
    B, N, C = x.shape
    nt = N // QT
    x2 = x.reshape(N, C)

    q, k, v, pk = pl.pallas_call(
        _qkv_kernel,
        grid=(nt,),
        in_specs=[
            pl.BlockSpec((QT, C), lambda i: (i, 0)),
            pl.BlockSpec((C, 3 * C), lambda i: (0, 0)),
            pl.BlockSpec((1, 3 * C), lambda i: (0, 0)),
            pl.BlockSpec((1, D), lambda i: (0, 0)),
            pl.BlockSpec((1, D), lambda i: (0, 0)),
        ],
        out_specs=[
            pl.BlockSpec((H, QT, D), lambda i: (0, i, 0)),
            pl.BlockSpec((H, QT, D), lambda i: (0, i, 0)),
            pl.BlockSpec((H, QT, D), lambda i: (0, i, 0)),
            pl.BlockSpec((H, QT, D), lambda i: (0, i, 0)),
        ],
        out_shape=[jax.ShapeDtypeStruct((H, N, D), jnp.float32)] * 4,
    )(x2, W_qkv, b_qkv.reshape(1, 3 * C), q_norm_w.reshape(1, D),
      k_norm_w.reshape(1, D))

    nc = N // CR
    k_sel = max(1, int(math.ceil(TOPK_RATIO * nc)))
    kp = k.reshape(H, nc, CR, D).transpose(0, 2, 1, 3).reshape(H, N, D)
    vp = v.reshape(H, nc, CR, D).transpose(0, 2, 1, 3).reshape(H, N, D)
    pkp = pk.reshape(H, nc, CR, D).transpose(0, 2, 1, 3).reshape(H, N, D)

    out = pl.pallas_call(
        functools.partial(_attn_kernel, n=N, k_sel=k_sel),
        grid=(nt, H),
        in_specs=[
            pl.BlockSpec((1, QT, D), lambda i, h: (h, i, 0)),
            pl.BlockSpec((1, N, D), lambda i, h: (h, 0, 0)),
            pl.BlockSpec((1, N, D), lambda i, h: (h, 0, 0)),
            pl.BlockSpec((1, N, D), lambda i, h: (h, 0, 0)),
            pl.BlockSpec((H, D, C), lambda i, h: (0, 0, 0)),
            pl.BlockSpec((1, C), lambda i, h: (0, 0)),
        ],
        out_specs=pl.BlockSpec((QT, C), lambda i, h: (i, 0)),
        out_shape=jax.ShapeDtypeStruct((N, C), jnp.float32),
        scratch_shapes=[pltpu.VMEM((H, QT, D), jnp.float32)],
        compiler_params=pltpu.CompilerParams(
            dimension_semantics=("arbitrary", "arbitrary")),
    )(q, kp, vp, pkp, W_proj.reshape(H, D, C), b_proj.reshape(1, C))

    return out.reshape(B, N, C)


# speedup vs baseline: 1.2257x; 1.0201x over previous
import functools
import math

import jax
import jax.numpy as jnp
from jax.experimental import pallas as pl
from jax.experimental.pallas import tpu as pltpu

H = 12
D = 64
CR = 8
TOPK_RATIO = 0.05
QT = 512


def _softmax_rows(t):
    m = jnp.max(t, axis=1, keepdims=True)
    e = jnp.exp(t - m)
    return e / jnp.sum(e, axis=1, keepdims=True)


def _qkv_kernel(x_ref, w_ref, b_ref, qnw_ref, knw_ref, pm_ref, ps_ref,
                q_ref, k_ref, v_ref, pk_ref):
    C = x_ref.shape[1]
    y = jnp.dot(x_ref[...], w_ref[...], preferred_element_type=jnp.float32)
    y = y + b_ref[...]
    yq = y[:, :C]
    yk = y[:, C:2 * C]
    del pm_ref, ps_ref
    nq = yq.shape[0]

    def ln3(t, w):
        t3 = t.reshape(nq, H, D)
        mu = jnp.mean(t3, axis=2, keepdims=True)
        d = t3 - mu
        var = jnp.mean(d * d, axis=2, keepdims=True)
        return (d * jax.lax.rsqrt(var + 1e-6)).reshape(nq, C) * w

    qn = ln3(yq, qnw_ref[...])
    kn = ln3(yk, knw_ref[...])
    e3 = jnp.exp(kn - 8.0).reshape(nq, H, D)
    pk = (e3 / jnp.sum(e3, axis=2, keepdims=True)).reshape(nq, C)
    for h in range(H):
        q_ref[h, :, :] = qn[:, h * D:(h + 1) * D]
        k_ref[h, :, :] = kn[:, h * D:(h + 1) * D]
        v_ref[h, :, :] = y[:, 2 * C + h * D:2 * C + (h + 1) * D]
        pk_ref[h, :, :] = pk[:, h * D:(h + 1) * D]


def _attn_kernel(q_ref, k_ref, v_ref, pk_ref, wp_ref, bp_ref, out_ref,
                 acc_ref, *, n, k_sel):
    h = pl.program_id(1)
    nc = n // CR
    scale = D ** -0.5
    q = q_ref[0]
    k = k_ref[0]
    v = v_ref[0]
    phi_k = pk_ref[0]

    kc = k[0:nc, :]
    for r in range(1, CR):
        kc = kc + k[r * nc:(r + 1) * nc, :]
    kc = kc * (1.0 / CR)

    dn = (((1,), (1,)), ((), ()))
    qs = q * scale
    router = jax.lax.dot_general(qs, kc, dn,
                                 preferred_element_type=jnp.float32)

    vals = router
    thresh = jnp.full((QT, 1), jnp.inf, jnp.float32)
    need = jnp.full((QT, 1), float(k_sel), jnp.float32)
    for _ in range(k_sel):
        m = jnp.max(vals, axis=1, keepdims=True)
        eq = vals == m
        c = jnp.sum(jnp.where(eq, 1.0, 0.0), axis=1, keepdims=True)
        thresh = jnp.minimum(thresh, jnp.where(need > 0.0, m, jnp.inf))
        need = need - c
        vals = jnp.where(eq, -jnp.inf, vals)

    bm = router >= thresh
    mask = jnp.concatenate([bm] * CR, axis=1)

    s = jax.lax.dot_general(qs, k, dn, preferred_element_type=jnp.float32)
    s = jnp.where(mask, s, -1e9)
    sm = jnp.max(s, axis=1, keepdims=True)
    p = jnp.exp(s - sm)
    o_sp = jnp.dot(p, v, preferred_element_type=jnp.float32)
    o_sp = o_sp / jnp.sum(p, axis=1, keepdims=True)

    phi_q = _softmax_rows(q)
    wl = jax.lax.dot_general(phi_q, phi_k, dn,
                             preferred_element_type=jnp.float32)
    wl = jnp.where(mask, 0.0, wl)
    den = jnp.sum(wl, axis=1, keepdims=True) + 1e-6
    o_lin = jnp.dot(wl, v, preferred_element_type=jnp.float32) / den

    acc_ref[h] = o_sp + o_lin

    @pl.when(h == H - 1)
    def _():
        res = bp_ref[...]
        for hh in range(H):
            res = res + jnp.dot(acc_ref[hh], wp_ref[hh],
                                preferred_element_type=jnp.float32)
        out_ref[...] = res


def kernel(x, W_qkv, b_qkv, q_norm_w, k_norm_w, W_proj, b_proj):
    B, N, C = x.shape
    nt = N // QT
    x2 = x.reshape(N, C)
    idx = jnp.arange(C) // D
    blk = idx[:, None] == idx[None, :]

    q, k, v, pk = pl.pallas_call(
        _qkv_kernel,
        grid=(nt,),
        in_specs=[
            pl.BlockSpec((QT, C), lambda i: (i, 0)),
            pl.BlockSpec((C, 3 * C), lambda i: (0, 0)),
            pl.BlockSpec((1, 3 * C), lambda i: (0, 0)),
            pl.BlockSpec((1, C), lambda i: (0, 0)),
            pl.BlockSpec((1, C), lambda i: (0, 0)),
            pl.BlockSpec((C, C), lambda i: (0, 0)),
            pl.BlockSpec((C, C), lambda i: (0, 0)),
        ],
        out_specs=[
            pl.BlockSpec((H, QT, D), lambda i: (0, i, 0)),
            pl.BlockSpec((H, QT, D), lambda i: (0, i, 0)),
            pl.BlockSpec((H, QT, D), lambda i: (0, i, 0)),
            pl.BlockSpec((H, QT, D), lambda i: (0, i, 0)),
        ],
        out_shape=[jax.ShapeDtypeStruct((H, N, D), jnp.float32)] * 4,
    )(x2, W_qkv, b_qkv.reshape(1, 3 * C),
      jnp.tile(q_norm_w, H).reshape(1, C),
      jnp.tile(k_norm_w, H).reshape(1, C),
      blk.astype(jnp.float32) / D, blk.astype(jnp.float32))

    nc = N // CR
    k_sel = max(1, int(math.ceil(TOPK_RATIO * nc)))
    kp = k.reshape(H, nc, CR, D).transpose(0, 2, 1, 3).reshape(H, N, D)
    vp = v.reshape(H, nc, CR, D).transpose(0, 2, 1, 3).reshape(H, N, D)
    pkp = pk.reshape(H, nc, CR, D).transpose(0, 2, 1, 3).reshape(H, N, D)

    out = pl.pallas_call(
        functools.partial(_attn_kernel, n=N, k_sel=k_sel),
        grid=(nt, H),
        in_specs=[
            pl.BlockSpec((1, QT, D), lambda i, h: (h, i, 0)),
            pl.BlockSpec((1, N, D), lambda i, h: (h, 0, 0)),
            pl.BlockSpec((1, N, D), lambda i, h: (h, 0, 0)),
            pl.BlockSpec((1, N, D), lambda i, h: (h, 0, 0)),
            pl.BlockSpec((H, D, C), lambda i, h: (0, 0, 0)),
            pl.BlockSpec((1, C), lambda i, h: (0, 0)),
        ],
        out_specs=pl.BlockSpec((QT, C), lambda i, h: (i, 0)),
        out_shape=jax.ShapeDtypeStruct((N, C), jnp.float32),
        scratch_shapes=[pltpu.VMEM((H, QT, D), jnp.float32)],
        compiler_params=pltpu.CompilerParams(
            dimension_semantics=("arbitrary", "arbitrary")),
    )(q, kp, vp, pkp, W_proj.reshape(H, D, C), b_proj.reshape(1, C))

    return out.reshape(B, N, C)


# speedup vs baseline: 1.2798x; 1.0441x over previous
import functools
import math

import jax
import jax.numpy as jnp
from jax.experimental import pallas as pl
from jax.experimental.pallas import tpu as pltpu

H = 12
D = 64
CR = 8
TOPK_RATIO = 0.05
QT = 512


def _softmax_rows(t):
    m = jnp.max(t, axis=1, keepdims=True)
    e = jnp.exp(t - m)
    return e / jnp.sum(e, axis=1, keepdims=True)


def _qkv_kernel(x_ref, w_ref, b_ref, qnw_ref, knw_ref,
                q_ref, k_ref, v_ref, pk_ref):
    C = x_ref.shape[1]
    y = jnp.dot(x_ref[...], w_ref[...], preferred_element_type=jnp.float32)
    y = y + b_ref[...]
    yq = y[:, :C]
    yk = y[:, C:2 * C]
    nq = yq.shape[0]

    def ln3(t, w):
        t3 = t.reshape(nq, H, D)
        mu = jnp.mean(t3, axis=2, keepdims=True)
        d = t3 - mu
        var = jnp.mean(d * d, axis=2, keepdims=True)
        return (d * jax.lax.rsqrt(var + 1e-6)).reshape(nq, C) * w

    qn = ln3(yq, qnw_ref[...])
    kn = ln3(yk, knw_ref[...])
    e3 = jnp.exp(kn - 8.0).reshape(nq, H, D)
    pk = (e3 / jnp.sum(e3, axis=2, keepdims=True)).reshape(nq, C)
    for h in range(H):
        q_ref[h, :, :] = qn[:, h * D:(h + 1) * D]
        k_ref[h, :, :] = kn[:, h * D:(h + 1) * D]
        v_ref[h, :, :] = y[:, 2 * C + h * D:2 * C + (h + 1) * D]
        pk_ref[h, :, :] = pk[:, h * D:(h + 1) * D]


def _attn_kernel(q_ref, k_ref, v_ref, pk_ref, wp_ref, bp_ref, out_ref,
                 acc_ref, *, n, k_sel):
    h = pl.program_id(1)
    nc = n // CR
    scale = D ** -0.5
    q = q_ref[0]
    k = k_ref[0]
    v = v_ref[0]
    phi_k = pk_ref[0]

    kc = k[0:nc, :]
    for r in range(1, CR):
        kc = kc + k[r * nc:(r + 1) * nc, :]
    kc = kc * (1.0 / CR)

    dn = (((1,), (1,)), ((), ()))
    qs = q * scale
    router = jax.lax.dot_general(qs, kc, dn,
                                 preferred_element_type=jnp.float32)

    vals = router
    thresh = jnp.full((QT, 1), jnp.inf, jnp.float32)
    need = jnp.full((QT, 1), float(k_sel), jnp.float32)
    for _ in range(k_sel):
        m = jnp.max(vals, axis=1, keepdims=True)
        eq = vals == m
        c = jnp.sum(jnp.where(eq, 1.0, 0.0), axis=1, keepdims=True)
        thresh = jnp.minimum(thresh, jnp.where(need > 0.0, m, jnp.inf))
        need = need - c
        vals = jnp.where(eq, -jnp.inf, vals)

    bm = router >= thresh
    mask = jnp.concatenate([bm] * CR, axis=1)

    vo = jnp.concatenate([v, jnp.ones((n, D), jnp.float32)], axis=1)

    s = jax.lax.dot_general(qs, k, dn, preferred_element_type=jnp.float32)
    s = jnp.where(mask, s, -1e9)
    sm = jnp.max(s, axis=1, keepdims=True)
    p = jnp.exp(s - sm)
    oc = jnp.dot(p, vo, preferred_element_type=jnp.float32)
    o_sp = oc[:, :D] / oc[:, D:D + 1]

    phi_q = _softmax_rows(q)
    wl = jax.lax.dot_general(phi_q, phi_k, dn,
                             preferred_element_type=jnp.float32)
    wl = jnp.where(mask, 0.0, wl)
    lc = jnp.dot(wl, vo, preferred_element_type=jnp.float32)
    o_lin = lc[:, :D] / (lc[:, D:D + 1] + 1e-6)

    acc_ref[h] = o_sp + o_lin

    @pl.when(h == H - 1)
    def _():
        res = bp_ref[...]
        for hh in range(H):
            res = res + jnp.dot(acc_ref[hh], wp_ref[hh],
                                preferred_element_type=jnp.float32)
        out_ref[...] = res


def kernel(x, W_qkv, b_qkv, q_norm_w, k_norm_w, W_proj, b_proj):
    B, N, C = x.shape
    nt = N // QT
    x2 = x.reshape(N, C)

    q, k, v, pk = pl.pallas_call(
        _qkv_kernel,
        grid=(nt,),
        in_specs=[
            pl.BlockSpec((QT, C), lambda i: (i, 0)),
            pl.BlockSpec((C, 3 * C), lambda i: (0, 0)),
            pl.BlockSpec((1, 3 * C), lambda i: (0, 0)),
            pl.BlockSpec((1, C), lambda i: (0, 0)),
            pl.BlockSpec((1, C), lambda i: (0, 0)),
        ],
        out_specs=[
            pl.BlockSpec((H, QT, D), lambda i: (0, i, 0)),
            pl.BlockSpec((H, QT, D), lambda i: (0, i, 0)),
            pl.BlockSpec((H, QT, D), lambda i: (0, i, 0)),
            pl.BlockSpec((H, QT, D), lambda i: (0, i, 0)),
        ],
        out_shape=[jax.ShapeDtypeStruct((H, N, D), jnp.float32)] * 4,
    )(x2, W_qkv, b_qkv.reshape(1, 3 * C),
      jnp.tile(q_norm_w, H).reshape(1, C),
      jnp.tile(k_norm_w, H).reshape(1, C))

    nc = N // CR
    k_sel = max(1, int(math.ceil(TOPK_RATIO * nc)))
    kp = k.reshape(H, nc, CR, D).transpose(0, 2, 1, 3).reshape(H, N, D)
    vp = v.reshape(H, nc, CR, D).transpose(0, 2, 1, 3).reshape(H, N, D)
    pkp = pk.reshape(H, nc, CR, D).transpose(0, 2, 1, 3).reshape(H, N, D)

    out = pl.pallas_call(
        functools.partial(_attn_kernel, n=N, k_sel=k_sel),
        grid=(nt, H),
        in_specs=[
            pl.BlockSpec((1, QT, D), lambda i, h: (h, i, 0)),
            pl.BlockSpec((1, N, D), lambda i, h: (h, 0, 0)),
            pl.BlockSpec((1, N, D), lambda i, h: (h, 0, 0)),
            pl.BlockSpec((1, N, D), lambda i, h: (h, 0, 0)),
            pl.BlockSpec((H, D, C), lambda i, h: (0, 0, 0)),
            pl.BlockSpec((1, C), lambda i, h: (0, 0)),
        ],
        out_specs=pl.BlockSpec((QT, C), lambda i, h: (i, 0)),
        out_shape=jax.ShapeDtypeStruct((N, C), jnp.float32),
        scratch_shapes=[pltpu.VMEM((H, QT, D), jnp.float32)],
        compiler_params=pltpu.CompilerParams(
            dimension_semantics=("arbitrary", "arbitrary")),
    )(q, kp, vp, pkp, W_proj.reshape(H, D, C), b_proj.reshape(1, C))

    return out.reshape(B, N, C)


# speedup vs baseline: 1.3475x; 1.0529x over previous
import functools
import math

import jax
import jax.numpy as jnp
from jax.experimental import pallas as pl
from jax.experimental.pallas import tpu as pltpu

H = 12
D = 64
CR = 8
TOPK_RATIO = 0.05
QT = 512


def _softmax_rows(t):
    m = jnp.max(t, axis=1, keepdims=True)
    e = jnp.exp(t - m)
    return e / jnp.sum(e, axis=1, keepdims=True)


def _qkv_kernel(x_ref, w_ref, b_ref, qnw_ref, knw_ref,
                q_ref, k_ref, v_ref, pk_ref):
    C = x_ref.shape[1]
    y = jnp.dot(x_ref[...], w_ref[...], preferred_element_type=jnp.float32)
    y = y + b_ref[...]
    yq = y[:, :C]
    yk = y[:, C:2 * C]
    nq = yq.shape[0]

    def ln3(t, w):
        t3 = t.reshape(nq, H, D)
        mu = jnp.mean(t3, axis=2, keepdims=True)
        d = t3 - mu
        var = jnp.mean(d * d, axis=2, keepdims=True)
        return (d * jax.lax.rsqrt(var + 1e-6)).reshape(nq, C) * w

    qn = ln3(yq, qnw_ref[...])
    kn = ln3(yk, knw_ref[...])
    e3 = jnp.exp(kn - 8.0).reshape(nq, H, D)
    pk = (e3 / jnp.sum(e3, axis=2, keepdims=True)).reshape(nq, C)
    for h in range(H):
        q_ref[h, :, :] = qn[:, h * D:(h + 1) * D]
        k_ref[h, :, :] = kn[:, h * D:(h + 1) * D]
        v_ref[h, :, :] = y[:, 2 * C + h * D:2 * C + (h + 1) * D]
        pk_ref[h, :, :] = pk[:, h * D:(h + 1) * D]


def _attn_kernel(q_ref, k_ref, v_ref, pk_ref, wp_ref, bp_ref, out_ref,
                 acc_ref, *, n, k_sel):
    h = pl.program_id(1)
    nc = n // CR
    scale = D ** -0.5
    q = q_ref[0]
    k = k_ref[0]
    v = v_ref[0]
    phi_k = pk_ref[0]

    kc = k[0:nc, :]
    for r in range(1, CR):
        kc = kc + k[r * nc:(r + 1) * nc, :]
    kc = kc * (1.0 / CR)

    dn = (((1,), (1,)), ((), ()))
    qs = q * scale
    router = jax.lax.dot_general(qs, kc, dn,
                                 preferred_element_type=jnp.float32)

    vals = router
    thresh = jnp.full((QT, 1), jnp.inf, jnp.float32)
    need = jnp.full((QT, 1), float(k_sel), jnp.float32)
    for _ in range(k_sel):
        m = jnp.max(vals, axis=1, keepdims=True)
        eq = vals == m
        c = jnp.sum(jnp.where(eq, 1.0, 0.0), axis=1, keepdims=True)
        thresh = jnp.minimum(thresh, jnp.where(need > 0.0, m, jnp.inf))
        need = need - c
        vals = jnp.where(eq, -jnp.inf, vals)

    bm = router >= thresh
    mask = jnp.concatenate([bm] * CR, axis=1)

    vo = jnp.concatenate([v, jnp.ones((n, D), jnp.float32)], axis=1)

    s = jax.lax.dot_general(qs, k, dn, preferred_element_type=jnp.float32)
    s = jnp.where(mask, s, -1e9)
    p = jnp.exp(s - 8.0)
    oc = jnp.dot(p, vo, preferred_element_type=jnp.float32)
    o_sp = oc[:, :D] / oc[:, D:D + 1]

    phi_q = _softmax_rows(q)
    wl = jax.lax.dot_general(phi_q, phi_k, dn,
                             preferred_element_type=jnp.float32)
    wl = jnp.where(mask, 0.0, wl)
    lc = jnp.dot(wl, vo, preferred_element_type=jnp.float32)
    o_lin = lc[:, :D] / (lc[:, D:D + 1] + 1e-6)

    acc_ref[h] = o_sp + o_lin

    @pl.when(h == H - 1)
    def _():
        res = bp_ref[...]
        for hh in range(H):
            res = res + jnp.dot(acc_ref[hh], wp_ref[hh],
                                preferred_element_type=jnp.float32)
        out_ref[...] = res


def kernel(x, W_qkv, b_qkv, q_norm_w, k_norm_w, W_proj, b_proj):
    B, N, C = x.shape
    nt = N // QT
    x2 = x.reshape(N, C)

    q, k, v, pk = pl.pallas_call(
        _qkv_kernel,
        grid=(nt,),
        in_specs=[
            pl.BlockSpec((QT, C), lambda i: (i, 0)),
            pl.BlockSpec((C, 3 * C), lambda i: (0, 0)),
            pl.BlockSpec((1, 3 * C), lambda i: (0, 0)),
            pl.BlockSpec((1, C), lambda i: (0, 0)),
            pl.BlockSpec((1, C), lambda i: (0, 0)),
        ],
        out_specs=[
            pl.BlockSpec((H, QT, D), lambda i: (0, i, 0)),
            pl.BlockSpec((H, QT, D), lambda i: (0, i, 0)),
            pl.BlockSpec((H, QT, D), lambda i: (0, i, 0)),
            pl.BlockSpec((H, QT, D), lambda i: (0, i, 0)),
        ],
        out_shape=[jax.ShapeDtypeStruct((H, N, D), jnp.float32)] * 4,
    )(x2, W_qkv, b_qkv.reshape(1, 3 * C),
      jnp.tile(q_norm_w, H).reshape(1, C),
      jnp.tile(k_norm_w, H).reshape(1, C))

    nc = N // CR
    k_sel = max(1, int(math.ceil(TOPK_RATIO * nc)))
    kp = k.reshape(H, nc, CR, D).transpose(0, 2, 1, 3).reshape(H, N, D)
    vp = v.reshape(H, nc, CR, D).transpose(0, 2, 1, 3).reshape(H, N, D)
    pkp = pk.reshape(H, nc, CR, D).transpose(0, 2, 1, 3).reshape(H, N, D)

    out = pl.pallas_call(
        functools.partial(_attn_kernel, n=N, k_sel=k_sel),
        grid=(nt, H),
        in_specs=[
            pl.BlockSpec((1, QT, D), lambda i, h: (h, i, 0)),
            pl.BlockSpec((1, N, D), lambda i, h: (h, 0, 0)),
            pl.BlockSpec((1, N, D), lambda i, h: (h, 0, 0)),
            pl.BlockSpec((1, N, D), lambda i, h: (h, 0, 0)),
            pl.BlockSpec((H, D, C), lambda i, h: (0, 0, 0)),
            pl.BlockSpec((1, C), lambda i, h: (0, 0)),
        ],
        out_specs=pl.BlockSpec((QT, C), lambda i, h: (i, 0)),
        out_shape=jax.ShapeDtypeStruct((N, C), jnp.float32),
        scratch_shapes=[pltpu.VMEM((H, QT, D), jnp.float32)],
        compiler_params=pltpu.CompilerParams(
            dimension_semantics=("arbitrary", "arbitrary")),
    )(q, kp, vp, pkp, W_proj.reshape(H, D, C), b_proj.reshape(1, C))

    return out.reshape(B, N, C)


# speedup vs baseline: 1.5275x; 1.1336x over previous
import functools
import math

import jax
import jax.numpy as jnp
from jax.experimental import pallas as pl
from jax.experimental.pallas import tpu as pltpu

H = 12
D = 64
CR = 8
TOPK_RATIO = 0.05
QT = 512


def _softmax_rows(t):
    m = jnp.max(t, axis=1, keepdims=True)
    e = jnp.exp(t - m)
    return e / jnp.sum(e, axis=1, keepdims=True)


def _qkv_kernel(x_ref, w_ref, b_ref, qnw_ref, knw_ref,
                q_ref, k_ref, v_ref, pk_ref):
    C = x_ref.shape[1]
    y = jnp.dot(x_ref[...], w_ref[...], preferred_element_type=jnp.float32)
    y = y + b_ref[...]
    yq = y[:, :C]
    yk = y[:, C:2 * C]
    nq = yq.shape[0]

    def ln3(t, w):
        t3 = t.reshape(nq, H, D)
        mu = jnp.mean(t3, axis=2, keepdims=True)
        d = t3 - mu
        var = jnp.mean(d * d, axis=2, keepdims=True)
        return (d * jax.lax.rsqrt(var + 1e-6)).reshape(nq, C) * w

    qn = ln3(yq, qnw_ref[...])
    kn = ln3(yk, knw_ref[...])
    e3 = jnp.exp(kn - 8.0).reshape(nq, H, D)
    pk = (e3 / jnp.sum(e3, axis=2, keepdims=True)).reshape(nq, C)
    nb = nq // CR
    for h in range(H):
        q_ref[h, :, :] = qn[:, h * D:(h + 1) * D]
        k_ref[h, :, :, :] = kn[:, h * D:(h + 1) * D].reshape(
            nb, CR, D).swapaxes(0, 1)
        v_ref[h, :, :, :] = y[:, 2 * C + h * D:2 * C + (h + 1) * D].reshape(
            nb, CR, D).swapaxes(0, 1)
        pk_ref[h, :, :, :] = pk[:, h * D:(h + 1) * D].reshape(
            nb, CR, D).swapaxes(0, 1)


def _attn_kernel(q_ref, k_ref, v_ref, pk_ref, wp_ref, bp_ref, out_ref,
                 acc_ref, *, n, k_sel):
    h = pl.program_id(1)
    nc = n // CR
    scale = D ** -0.5
    q = q_ref[0]
    k = k_ref[0].reshape(n, D)
    v = v_ref[0].reshape(n, D)
    phi_k = pk_ref[0].reshape(n, D)

    kc = k[0:nc, :]
    for r in range(1, CR):
        kc = kc + k[r * nc:(r + 1) * nc, :]
    kc = kc * (1.0 / CR)

    dn = (((1,), (1,)), ((), ()))
    qs = q * scale
    router = jax.lax.dot_general(qs, kc, dn,
                                 preferred_element_type=jnp.float32)

    vals = router
    thresh = jnp.full((QT, 1), jnp.inf, jnp.float32)
    need = jnp.full((QT, 1), float(k_sel), jnp.float32)
    for _ in range(k_sel):
        m = jnp.max(vals, axis=1, keepdims=True)
        eq = vals == m
        c = jnp.sum(jnp.where(eq, 1.0, 0.0), axis=1, keepdims=True)
        thresh = jnp.minimum(thresh, jnp.where(need > 0.0, m, jnp.inf))
        need = need - c
        vals = jnp.where(eq, -jnp.inf, vals)

    bm = router >= thresh
    mask = jnp.concatenate([bm] * CR, axis=1)

    vo = jnp.concatenate([v, jnp.ones((n, D), jnp.float32)], axis=1)

    s = jax.lax.dot_general(qs, k, dn, preferred_element_type=jnp.float32)
    s = jnp.where(mask, s, -1e9)
    p = jnp.exp(s - 8.0)
    oc = jnp.dot(p, vo, preferred_element_type=jnp.float32)
    o_sp = oc[:, :D] / oc[:, D:D + 1]

    phi_q = _softmax_rows(q)
    wl = jax.lax.dot_general(phi_q, phi_k, dn,
                             preferred_element_type=jnp.float32)
    wl = jnp.where(mask, 0.0, wl)
    lc = jnp.dot(wl, vo, preferred_element_type=jnp.float32)
    o_lin = lc[:, :D] / (lc[:, D:D + 1] + 1e-6)

    acc_ref[h] = o_sp + o_lin

    @pl.when(h == H - 1)
    def _():
        res = bp_ref[...]
        for hh in range(H):
            res = res + jnp.dot(acc_ref[hh], wp_ref[hh],
                                preferred_element_type=jnp.float32)
        out_ref[...] = res


def kernel(x, W_qkv, b_qkv, q_norm_w, k_norm_w, W_proj, b_proj):
    B, N, C = x.shape
    nt = N // QT
    x2 = x.reshape(N, C)

    nc = N // CR
    qt_b = QT // CR
    out_specs = [
            pl.BlockSpec((H, QT, D), lambda i: (0, i, 0)),
            pl.BlockSpec((H, CR, qt_b, D), lambda i: (0, 0, i, 0)),
            pl.BlockSpec((H, CR, qt_b, D), lambda i: (0, 0, i, 0)),
            pl.BlockSpec((H, CR, qt_b, D), lambda i: (0, 0, i, 0)),
    ]
    q, kp, vp, pkp = pl.pallas_call(
        _qkv_kernel,
        grid=(nt,),
        in_specs=[
            pl.BlockSpec((QT, C), lambda i: (i, 0)),
            pl.BlockSpec((C, 3 * C), lambda i: (0, 0)),
            pl.BlockSpec((1, 3 * C), lambda i: (0, 0)),
            pl.BlockSpec((1, C), lambda i: (0, 0)),
            pl.BlockSpec((1, C), lambda i: (0, 0)),
        ],
        out_specs=out_specs,
        out_shape=[jax.ShapeDtypeStruct((H, N, D), jnp.float32)] +
                  [jax.ShapeDtypeStruct((H, CR, nc, D), jnp.float32)] * 3,
    )(x2, W_qkv, b_qkv.reshape(1, 3 * C),
      jnp.tile(q_norm_w, H).reshape(1, C),
      jnp.tile(k_norm_w, H).reshape(1, C))

    k_sel = max(1, int(math.ceil(TOPK_RATIO * nc)))

    out = pl.pallas_call(
        functools.partial(_attn_kernel, n=N, k_sel=k_sel),
        grid=(nt, H),
        in_specs=[
            pl.BlockSpec((1, QT, D), lambda i, h: (h, i, 0)),
            pl.BlockSpec((1, CR, nc, D), lambda i, h: (h, 0, 0, 0)),
            pl.BlockSpec((1, CR, nc, D), lambda i, h: (h, 0, 0, 0)),
            pl.BlockSpec((1, CR, nc, D), lambda i, h: (h, 0, 0, 0)),
            pl.BlockSpec((H, D, C), lambda i, h: (0, 0, 0)),
            pl.BlockSpec((1, C), lambda i, h: (0, 0)),
        ],
        out_specs=pl.BlockSpec((QT, C), lambda i, h: (i, 0)),
        out_shape=jax.ShapeDtypeStruct((N, C), jnp.float32),
        scratch_shapes=[pltpu.VMEM((H, QT, D), jnp.float32)],
        compiler_params=pltpu.CompilerParams(
            dimension_semantics=("arbitrary", "arbitrary")),
    )(q, kp, vp, pkp, W_proj.reshape(H, D, C), b_proj.reshape(1, C))

    return out.reshape(B, N, C)


# speedup vs baseline: 1.5278x; 1.0002x over previous
import functools
import math

import jax
import jax.numpy as jnp
from jax.experimental import pallas as pl
from jax.experimental.pallas import tpu as pltpu

H = 12
D = 64
CR = 8
TOPK_RATIO = 0.05
QT = 512


def _softmax_rows(t):
    m = jnp.max(t, axis=1, keepdims=True)
    e = jnp.exp(t - m)
    return e / jnp.sum(e, axis=1, keepdims=True)


def _qkv_kernel(x_ref, w_ref, b_ref, qnw_ref, knw_ref,
                q_ref, k_ref, v_ref, pk_ref):
    C = x_ref.shape[1]
    y = jnp.dot(x_ref[...], w_ref[...], preferred_element_type=jnp.float32)
    y = y + b_ref[...]
    yq = y[:, :C]
    yk = y[:, C:2 * C]
    nq = yq.shape[0]

    def ln3(t, w):
        t3 = t.reshape(nq, H, D)
        mu = jnp.mean(t3, axis=2, keepdims=True)
        d = t3 - mu
        var = jnp.mean(d * d, axis=2, keepdims=True)
        return (d * jax.lax.rsqrt(var + 1e-6)).reshape(nq, C) * w

    qn = ln3(yq, qnw_ref[...])
    kn = ln3(yk, knw_ref[...])
    e3 = jnp.exp(kn - 8.0).reshape(nq, H, D)
    pk = (e3 / jnp.sum(e3, axis=2, keepdims=True)).reshape(nq, C)
    nb = nq // CR
    for h in range(H):
        q_ref[h, :, :] = qn[:, h * D:(h + 1) * D]
        k_ref[h, :, :, :] = kn[:, h * D:(h + 1) * D].reshape(
            nb, CR, D).swapaxes(0, 1)
        v_ref[h, :, :, :] = y[:, 2 * C + h * D:2 * C + (h + 1) * D].reshape(
            nb, CR, D).swapaxes(0, 1)
        pk_ref[h, :, :, :] = pk[:, h * D:(h + 1) * D].reshape(
            nb, CR, D).swapaxes(0, 1)


def _attn_kernel(q_ref, k_ref, v_ref, pk_ref, wp_ref, bp_ref, out_ref,
                 acc_ref, *, n, k_sel):
    h = pl.program_id(1)
    nc = n // CR
    scale = D ** -0.5
    q = q_ref[0]
    k = k_ref[0].reshape(n, D)
    v = v_ref[0].reshape(n, D)
    phi_k = pk_ref[0].reshape(n, D)

    kc = k[0:nc, :]
    for r in range(1, CR):
        kc = kc + k[r * nc:(r + 1) * nc, :]
    kc = kc * (1.0 / CR)

    dn = (((1,), (1,)), ((), ()))
    qs = q * scale
    router = jax.lax.dot_general(qs, kc, dn,
                                 preferred_element_type=jnp.float32)

    vals = router
    thresh = jnp.full((QT, 1), jnp.inf, jnp.float32)
    need = jnp.full((QT, 1), float(k_sel), jnp.float32)
    for _ in range(k_sel):
        m = jnp.max(vals, axis=1, keepdims=True)
        eq = vals == m
        c = jnp.sum(jnp.where(eq, 1.0, 0.0), axis=1, keepdims=True)
        thresh = jnp.minimum(thresh, jnp.where(need > 0.0, m, jnp.inf))
        need = need - c
        vals = jnp.where(eq, -jnp.inf, vals)

    bm = router >= thresh
    mask = jnp.concatenate([bm] * CR, axis=1)

    vo = jnp.concatenate([v, jnp.ones((n, D), jnp.float32)], axis=1)

    s = jax.lax.dot_general(qs, k, dn, preferred_element_type=jnp.float32)
    s = jnp.where(mask, s, -1e9)
    p = jnp.exp(s - 8.0)
    oc = jnp.dot(p, vo, preferred_element_type=jnp.float32)
    o_sp = oc[:, :D] / oc[:, D:D + 1]

    phi_q = _softmax_rows(q)
    wl = jax.lax.dot_general(phi_q, phi_k, dn,
                             preferred_element_type=jnp.float32)
    wl = jnp.where(mask, 0.0, wl)
    lc = jnp.dot(wl, vo, preferred_element_type=jnp.float32)
    o_lin = lc[:, :D] / (lc[:, D:D + 1] + 1e-6)

    attn = o_sp + o_lin
    for hh in range(H):
        @pl.when(h == hh)
        def _(hh=hh):
            acc_ref[:, hh * D:(hh + 1) * D] = attn

    @pl.when(h == H - 1)
    def _():
        out_ref[...] = bp_ref[...] + jnp.dot(
            acc_ref[...], wp_ref[...], preferred_element_type=jnp.float32)


def kernel(x, W_qkv, b_qkv, q_norm_w, k_norm_w, W_proj, b_proj):
    B, N, C = x.shape
    nt = N // QT
    x2 = x.reshape(N, C)

    nc = N // CR
    qt_b = QT // CR
    out_specs = [
            pl.BlockSpec((H, QT, D), lambda i: (0, i, 0)),
            pl.BlockSpec((H, CR, qt_b, D), lambda i: (0, 0, i, 0)),
            pl.BlockSpec((H, CR, qt_b, D), lambda i: (0, 0, i, 0)),
            pl.BlockSpec((H, CR, qt_b, D), lambda i: (0, 0, i, 0)),
    ]
    q, kp, vp, pkp = pl.pallas_call(
        _qkv_kernel,
        grid=(nt,),
        in_specs=[
            pl.BlockSpec((QT, C), lambda i: (i, 0)),
            pl.BlockSpec((C, 3 * C), lambda i: (0, 0)),
            pl.BlockSpec((1, 3 * C), lambda i: (0, 0)),
            pl.BlockSpec((1, C), lambda i: (0, 0)),
            pl.BlockSpec((1, C), lambda i: (0, 0)),
        ],
        out_specs=out_specs,
        out_shape=[jax.ShapeDtypeStruct((H, N, D), jnp.float32)] +
                  [jax.ShapeDtypeStruct((H, CR, nc, D), jnp.float32)] * 3,
    )(x2, W_qkv, b_qkv.reshape(1, 3 * C),
      jnp.tile(q_norm_w, H).reshape(1, C),
      jnp.tile(k_norm_w, H).reshape(1, C))

    k_sel = max(1, int(math.ceil(TOPK_RATIO * nc)))

    out = pl.pallas_call(
        functools.partial(_attn_kernel, n=N, k_sel=k_sel),
        grid=(nt, H),
        in_specs=[
            pl.BlockSpec((1, QT, D), lambda i, h: (h, i, 0)),
            pl.BlockSpec((1, CR, nc, D), lambda i, h: (h, 0, 0, 0)),
            pl.BlockSpec((1, CR, nc, D), lambda i, h: (h, 0, 0, 0)),
            pl.BlockSpec((1, CR, nc, D), lambda i, h: (h, 0, 0, 0)),
            pl.BlockSpec((C, C), lambda i, h: (0, 0)),
            pl.BlockSpec((1, C), lambda i, h: (0, 0)),
        ],
        out_specs=pl.BlockSpec((QT, C), lambda i, h: (i, 0)),
        out_shape=jax.ShapeDtypeStruct((N, C), jnp.float32),
        scratch_shapes=[pltpu.VMEM((QT, C), jnp.float32)],
        compiler_params=pltpu.CompilerParams(
            dimension_semantics=("arbitrary", "arbitrary")),
    )(q, kp, vp, pkp, W_proj, b_proj.reshape(1, C))

    return out.reshape(B, N, C)
